# baseline (speedup 1.0000x reference)
import functools

import jax
import jax.numpy as jnp
from jax import lax
from jax.experimental import pallas as pl
from jax.experimental.pallas import tpu as pltpu
from jax.experimental.pallas import tpu_sc as plsc

N = 10000
E = 320000
D = 128
G = 32
EPS = 1e-5

NC = 2
NS = 16
L = 16

_MESH = plsc.VectorSubcoreMesh(core_axis_name="c", subcore_axis_name="s")


_C1 = 2000
_EP1 = E // (NC * NS)


@functools.partial(
    pl.kernel,
    out_type=jax.ShapeDtypeStruct((NC * 2 * N,), jnp.float32),
    mesh=_MESH,
    compiler_params=pltpu.CompilerParams(needs_layout_passes=False, use_tc_tiling_on_sc=False),
    scratch_types=[
        pltpu.VMEM((_C1,), jnp.int32),
        pltpu.VMEM((_C1,), jnp.int32),
        pltpu.VMEM((_C1,), jnp.float32),
        pltpu.VMEM((N,), jnp.float32),
        pltpu.VMEM((N,), jnp.float32),
        pltpu.VMEM((1280,), jnp.float32),
        pltpu.VMEM((1280,), jnp.float32),
        pltpu.VMEM_SHARED((NS, 2 * N), jnp.float32),
    ],
)
def _deg_kernel(src_hbm, dst_hbm, ew_hbm, out_hbm,
                src_v, dst_v, ew_v, dego_v, degi_v, accbuf, tmpbuf, stage_sh):
    c = lax.axis_index("c")
    s = lax.axis_index("s")

    def zero(j, carry):
        for i in range(5):
            o = pl.ds(j * 80 + i * L, L)
            dego_v[o] = jnp.zeros((L,), jnp.float32)
            degi_v[o] = jnp.zeros((L,), jnp.float32)
        return carry

    lax.fori_loop(0, N // 80, zero, 0)

    base = (c * NS + s) * _EP1

    def chunk(j, carry):
        off = base + j * _C1
        pltpu.sync_copy(src_hbm.at[pl.ds(off, _C1)], src_v)
        pltpu.sync_copy(dst_hbm.at[pl.ds(off, _C1)], dst_v)
        pltpu.sync_copy(ew_hbm.at[pl.ds(off, _C1)], ew_v)

        @plsc.parallel_loop(0, _C1 // L, unroll=8)
        def _(i):
            sl = pl.ds(i * L, L)
            wv = ew_v[sl]
            plsc.addupdate_scatter(dego_v, [src_v[sl]], wv)
            plsc.addupdate_scatter(degi_v, [dst_v[sl]], wv)

        return carry

    lax.fori_loop(0, _EP1 // _C1, chunk, 0)

    pltpu.sync_copy(dego_v, stage_sh.at[s, pl.ds(0, N)])
    pltpu.sync_copy(degi_v, stage_sh.at[s, pl.ds(N, N)])

    plsc.subcore_barrier()

    def reduce_drain(off, rows):
        for i in range(rows // L):
            accbuf[pl.ds(i * L, L)] = jnp.zeros((L,), jnp.float32)
        for k in range(NS):
            pltpu.sync_copy(stage_sh.at[k, pl.ds(off, rows)],
                            tmpbuf.at[pl.ds(0, rows)])
            for i in range(rows // L):
                sl = pl.ds(i * L, L)
                accbuf[sl] = accbuf[sl] + tmpbuf[sl]
        pltpu.sync_copy(accbuf.at[pl.ds(0, rows)],
                        out_hbm.at[pl.ds(c * 2 * N + off, rows)])

    @pl.when(s < NS - 1)
    def _():
        reduce_drain(s * 1248, 1248)

    @pl.when(s == NS - 1)
    def _():
        reduce_drain((NS - 1) * 1248, 1280)


def _safe_rsqrt(deg):
    deg_safe = jnp.where(deg > 0, deg, 1.0)
    return jnp.where(deg > 0, lax.rsqrt(deg_safe), 0.0)


def _dense_body(x_ref, gamma_ref, beta_ref, w_ref, degp_ref,
                featg_ref, normdst_ref):
    x = x_ref[...]
    mean = jnp.mean(x, axis=0, keepdims=True)
    xc = x - mean
    var = jnp.mean(xc * xc, axis=0, keepdims=True)
    h = xc * lax.rsqrt(var + EPS) * gamma_ref[...] + beta_ref[...]
    h = jnp.maximum(h, 0.0)
    dsum = jnp.sum(degp_ref[...], axis=0)
    norm_src = _safe_rsqrt(dsum[:N])
    featg_ref[...] = lax.dot_general(
        w_ref[...], h * norm_src, (((0,), (1,)), ((), ())),
        preferred_element_type=jnp.float32)
    normdst_ref[...] = _safe_rsqrt(dsum[N:])


_dense_call = pl.pallas_call(
    _dense_body,
    out_shape=[
        jax.ShapeDtypeStruct((G, N), jnp.float32),
        jax.ShapeDtypeStruct((N, 1), jnp.float32),
    ],
)


_C2 = 4000
_TPG = 4
_FPT = 4
_EP2 = E // _TPG


@functools.partial(
    pl.kernel,
    out_type=jax.ShapeDtypeStruct((N, G), jnp.float32),
    mesh=_MESH,
    compiler_params=pltpu.CompilerParams(needs_layout_passes=False, use_tc_tiling_on_sc=False),
    scratch_types=[
        [pltpu.VMEM((N,), jnp.float32)] * _FPT,
        [pltpu.VMEM((N,), jnp.float32)] * _FPT,
        pltpu.VMEM((_C2,), jnp.int32),
        pltpu.VMEM((_C2,), jnp.int32),
        pltpu.VMEM((_C2,), jnp.float32),
        pltpu.VMEM((640,), jnp.float32),
        pltpu.VMEM((G,), jnp.float32),
        pltpu.VMEM((NS * 640,), jnp.float32),
        pltpu.VMEM((640, 16), jnp.float32),
        pltpu.VMEM_SHARED((NS, N), jnp.float32),
    ],
)
def _agg_kernel(featg_hbm, normdst_hbm, b_hbm, src_hbm, dst_hbm, ew_hbm,
                out_hbm, feat_fs, acc_fs, src_v, dst_v, ew_v, nd_v, b_v,
                dbuf, outbuf, stage_sh):
    c = lax.axis_index("c")
    s = lax.axis_index("s")
    fg = c * 4 + s // _TPG
    slot = s % _TPG

    for f in range(_FPT):
        pltpu.sync_copy(featg_hbm.at[fg * _FPT + f], feat_fs[f])
    pltpu.sync_copy(b_hbm, b_v)

    def zero(j, carry):
        for f in range(_FPT):
            acc_fs[f][pl.ds(j * L, L)] = jnp.zeros((L,), jnp.float32)
        return carry

    lax.fori_loop(0, N // L, zero, 0)

    base = slot * _EP2

    def chunk(j, carry):
        off = base + j * _C2
        pltpu.sync_copy(src_hbm.at[pl.ds(off, _C2)], src_v)
        pltpu.sync_copy(dst_hbm.at[pl.ds(off, _C2)], dst_v)
        pltpu.sync_copy(ew_hbm.at[pl.ds(off, _C2)], ew_v)

        @plsc.parallel_loop(0, _C2 // L, unroll=16)
        def _(i):
            sl = pl.ds(i * L, L)
            sv = src_v[sl]
            dv = dst_v[sl]
            wv = ew_v[sl]
            for f in range(_FPT):
                vals = plsc.load_gather(feat_fs[f], [sv]) * wv
                plsc.addupdate_scatter(acc_fs[f], [dv], vals)

        return carry

    lax.fori_loop(0, _EP2 // _C2, chunk, 0)

    iota16 = lax.iota(jnp.int32, L)

    def copy_compute(r0, rows, f):
        for k in range(NS):
            pltpu.sync_copy(stage_sh.at[k, pl.ds(r0, rows)],
                            dbuf.at[pl.ds(k * 640, rows)])

        def body(i, carry):
            nd = nd_v[pl.ds(i * L, L)]
            for lg in range(_TPG):
                k0 = lg * _TPG
                q = [dbuf[pl.ds((k0 + k) * 640 + i * L, L)]
                     for k in range(_TPG)]
                v = (q[0] + q[1]) + (q[2] + q[3])
                fl = lg * _FPT + f
                bf = plsc.load_gather(
                    b_v, [jnp.full((L,), c * 16 + fl, jnp.int32)])
                plsc.store_scatter(outbuf,
                                   [iota16 + i * L,
                                    jnp.full((L,), fl, jnp.int32)],
                                   v * nd + bf)
            return carry

        lax.fori_loop(0, rows // L, body, 0)

    @pl.when(s < NS - 1)
    def _():
        pltpu.sync_copy(normdst_hbm.at[pl.ds(s * 624, 624)],
                        nd_v.at[pl.ds(0, 624)])

    @pl.when(s == NS - 1)
    def _():
        pltpu.sync_copy(normdst_hbm.at[pl.ds((NS - 1) * 624, 640)], nd_v)

    for f in range(_FPT):
        pltpu.sync_copy(acc_fs[f], stage_sh.at[s])
        plsc.subcore_barrier()

        @pl.when(s < NS - 1)
        def _():
            copy_compute(s * 624, 624, f)

        @pl.when(s == NS - 1)
        def _():
            copy_compute((NS - 1) * 624, 640, f)

        plsc.subcore_barrier()

    @pl.when(s < NS - 1)
    def _():
        pltpu.sync_copy(outbuf.at[pl.ds(0, 624)],
                        out_hbm.at[pl.ds(s * 624, 624), pl.ds(c * 16, 16)])

    @pl.when(s == NS - 1)
    def _():
        pltpu.sync_copy(outbuf,
                        out_hbm.at[pl.ds((NS - 1) * 624, 640),
                                   pl.ds(c * 16, 16)])


def kernel(x, edge_index, edge_weight, gamma, beta, W, b):
    src = edge_index[0]
    dst = edge_index[1]
    degp = _deg_kernel(src, dst, edge_weight)
    featg, normdst = _dense_call(
        x, gamma.reshape(1, D), beta.reshape(1, D), W,
        degp.reshape(NC, 2 * N, 1))
    return _agg_kernel(featg, normdst.reshape(N), b, src, dst,
                       edge_weight)


# speedup vs baseline: 1.1688x; 1.1688x over previous
import functools

import jax
import jax.numpy as jnp
from jax import lax
from jax.experimental import pallas as pl
from jax.experimental.pallas import tpu as pltpu
from jax.experimental.pallas import tpu_sc as plsc

N = 10000
E = 320000
D = 128
G = 32
EPS = 1e-5

NC = 2
NS = 16
L = 16

_MESH = plsc.VectorSubcoreMesh(core_axis_name="c", subcore_axis_name="s")


def _dense_body(x_ref, gamma_ref, beta_ref, w_ref, featg_ref):
    x = x_ref[...]
    mean = jnp.mean(x, axis=0, keepdims=True)
    xc = x - mean
    var = jnp.mean(xc * xc, axis=0, keepdims=True)
    h = xc * lax.rsqrt(var + EPS) * gamma_ref[...] + beta_ref[...]
    h = jnp.maximum(h, 0.0)
    featg_ref[...] = lax.dot_general(
        w_ref[...], h, (((0,), (1,)), ((), ())),
        preferred_element_type=jnp.float32)


_dense_call = pl.pallas_call(
    _dense_body,
    out_shape=jax.ShapeDtypeStruct((G, N), jnp.float32),
)


_C1 = 4000
_EPD = E // NS
_C2 = 4000
_TPG = 4
_FPT = 4
_EP2 = E // _TPG


def _rsqrt16(x):
    i = plsc.bitcast(x, jnp.int32)
    y = plsc.bitcast(jnp.full((L,), 0x5F3759DF, jnp.int32) - (i >> 1),
                     jnp.float32)
    for _ in range(3):
        y = y * (1.5 - 0.5 * x * y * y)
    return jnp.where(x > 0, y, 0.0)


@functools.partial(
    pl.kernel,
    out_type=jax.ShapeDtypeStruct((N, G), jnp.float32),
    mesh=_MESH,
    compiler_params=pltpu.CompilerParams(needs_layout_passes=False,
                                         use_tc_tiling_on_sc=False),
    scratch_types=[
        [pltpu.VMEM((N,), jnp.float32)] * _FPT,
        [pltpu.VMEM((N,), jnp.float32)] * _FPT,
        pltpu.VMEM((_C2,), jnp.int32),
        pltpu.VMEM((_C2,), jnp.int32),
        pltpu.VMEM((_C2,), jnp.float32),
        pltpu.VMEM((640,), jnp.float32),
        pltpu.VMEM((640,), jnp.float32),
        pltpu.VMEM((G,), jnp.float32),
        pltpu.VMEM((NS * 640,), jnp.float32),
        pltpu.VMEM((640, 16), jnp.float32),
        pltpu.VMEM_SHARED((NS, N), jnp.float32),
        pltpu.VMEM_SHARED((2 * N,), jnp.float32),
    ],
)
def _graph_kernel(featg_hbm, b_hbm, src_hbm, dst_hbm, ew_hbm, out_hbm,
                  feat_fs, acc_fs, src_v, dst_v, ew_v, nd_v, nbuf, b_v,
                  dbuf, outbuf, stage_sh, norm_sh):
    c = lax.axis_index("c")
    s = lax.axis_index("s")
    fg = c * 4 + s // _TPG
    slot = s % _TPG
    iota16 = lax.iota(jnp.int32, L)

    for f in range(_FPT):
        pltpu.sync_copy(featg_hbm.at[fg * _FPT + f], feat_fs[f])
    pltpu.sync_copy(b_hbm, b_v)

    def zero2(j, carry):
        o = pl.ds(j * L, L)
        acc_fs[0][o] = jnp.zeros((L,), jnp.float32)
        acc_fs[1][o] = jnp.zeros((L,), jnp.float32)
        return carry

    lax.fori_loop(0, N // L, zero2, 0)

    dbase = s * _EPD

    def dchunk(j, carry):
        off = dbase + j * _C1
        pltpu.sync_copy(src_hbm.at[pl.ds(off, _C1)], src_v)
        pltpu.sync_copy(dst_hbm.at[pl.ds(off, _C1)], dst_v)
        pltpu.sync_copy(ew_hbm.at[pl.ds(off, _C1)], ew_v)

        @plsc.parallel_loop(0, _C1 // L, unroll=16)
        def _(i):
            sl = pl.ds(i * L, L)
            wv = ew_v[sl]
            plsc.addupdate_scatter(acc_fs[0], [src_v[sl]], wv)
            plsc.addupdate_scatter(acc_fs[1], [dst_v[sl]], wv)

        return carry

    lax.fori_loop(0, _EPD // _C1, dchunk, 0)

    def norm_pass(which, r0, rows):
        for k in range(NS):
            pltpu.sync_copy(stage_sh.at[k, pl.ds(r0, rows)],
                            dbuf.at[pl.ds(k * 640, rows)])

        def body(i, carry):
            v = dbuf[pl.ds(i * L, L)]
            for k in range(1, NS):
                v = v + dbuf[pl.ds(k * 640 + i * L, L)]
            nbuf[pl.ds(i * L, L)] = _rsqrt16(v)
            return carry

        lax.fori_loop(0, rows // L, body, 0)
        pltpu.sync_copy(nbuf.at[pl.ds(0, rows)],
                        norm_sh.at[pl.ds(which * N + r0, rows)])

    for which in range(2):
        pltpu.sync_copy(acc_fs[which], stage_sh.at[s])
        plsc.subcore_barrier()

        @pl.when(s < NS - 1)
        def _():
            norm_pass(which, s * 624, 624)

        @pl.when(s == NS - 1)
        def _():
            norm_pass(which, (NS - 1) * 624, 640)

        plsc.subcore_barrier()

    pltpu.sync_copy(norm_sh.at[pl.ds(0, N)], acc_fs[2])

    def scale(j, carry):
        o = pl.ds(j * L, L)
        ns_ = acc_fs[2][o]
        for f in range(_FPT):
            feat_fs[f][o] = feat_fs[f][o] * ns_
        return carry

    lax.fori_loop(0, N // L, scale, 0)

    def zero4(j, carry):
        for f in range(_FPT):
            acc_fs[f][pl.ds(j * L, L)] = jnp.zeros((L,), jnp.float32)
        return carry

    lax.fori_loop(0, N // L, zero4, 0)

    base = slot * _EP2

    def chunk(j, carry):
        off = base + j * _C2
        pltpu.sync_copy(src_hbm.at[pl.ds(off, _C2)], src_v)
        pltpu.sync_copy(dst_hbm.at[pl.ds(off, _C2)], dst_v)
        pltpu.sync_copy(ew_hbm.at[pl.ds(off, _C2)], ew_v)

        @plsc.parallel_loop(0, _C2 // L, unroll=16)
        def _(i):
            sl = pl.ds(i * L, L)
            sv = src_v[sl]
            dv = dst_v[sl]
            wv = ew_v[sl]
            for f in range(_FPT):
                vals = plsc.load_gather(feat_fs[f], [sv]) * wv
                plsc.addupdate_scatter(acc_fs[f], [dv], vals)

        return carry

    lax.fori_loop(0, _EP2 // _C2, chunk, 0)

    def copy_compute(r0, rows, f):
        for k in range(NS):
            pltpu.sync_copy(stage_sh.at[k, pl.ds(r0, rows)],
                            dbuf.at[pl.ds(k * 640, rows)])

        def body(i, carry):
            nd = nd_v[pl.ds(i * L, L)]
            for lg in range(_TPG):
                k0 = lg * _TPG
                q = [dbuf[pl.ds((k0 + k) * 640 + i * L, L)]
                     for k in range(_TPG)]
                v = (q[0] + q[1]) + (q[2] + q[3])
                fl = lg * _FPT + f
                bf = plsc.load_gather(
                    b_v, [jnp.full((L,), c * 16 + fl, jnp.int32)])
                plsc.store_scatter(outbuf,
                                   [iota16 + i * L,
                                    jnp.full((L,), fl, jnp.int32)],
                                   v * nd + bf)
            return carry

        lax.fori_loop(0, rows // L, body, 0)

    @pl.when(s < NS - 1)
    def _():
        pltpu.sync_copy(norm_sh.at[pl.ds(N + s * 624, 624)],
                        nd_v.at[pl.ds(0, 624)])

    @pl.when(s == NS - 1)
    def _():
        pltpu.sync_copy(norm_sh.at[pl.ds(N + (NS - 1) * 624, 640)], nd_v)

    for f in range(_FPT):
        pltpu.sync_copy(acc_fs[f], stage_sh.at[s])
        plsc.subcore_barrier()

        @pl.when(s < NS - 1)
        def _():
            copy_compute(s * 624, 624, f)

        @pl.when(s == NS - 1)
        def _():
            copy_compute((NS - 1) * 624, 640, f)

        plsc.subcore_barrier()

    @pl.when(s < NS - 1)
    def _():
        pltpu.sync_copy(outbuf.at[pl.ds(0, 624)],
                        out_hbm.at[pl.ds(s * 624, 624), pl.ds(c * 16, 16)])

    @pl.when(s == NS - 1)
    def _():
        pltpu.sync_copy(outbuf,
                        out_hbm.at[pl.ds((NS - 1) * 624, 640),
                                   pl.ds(c * 16, 16)])


def kernel(x, edge_index, edge_weight, gamma, beta, W, b):
    src = edge_index[0]
    dst = edge_index[1]
    featg = _dense_call(x, gamma.reshape(1, D), beta.reshape(1, D), W)
    return _graph_kernel(featg, b, src, dst, edge_weight)
